# baseline (speedup 1.0000x reference)
import dataclasses
import functools

import jax
import jax.numpy as jnp
from jax import lax
from jax.experimental import pallas as pl
from jax.experimental.pallas import tpu as pltpu
from jax.experimental.pallas import tpu_sc as plsc

D = 64
KEY_V = 100000
VAL_V = 1000000
MAX_DEPTH = 64
MAX_SIB = 256
NODE_TYPES = 4
KIND_V = 1000
B = 4096
L = 200
EPS = 1e-5

NL = 16
NW = 32
CHN = 128
N = B * L
N2 = N // 2
CPW = N // (NW * CHN)
NCHUNK = N // CHN
NIDX = 9 * CHN
LNB = 4096


def _sc_embed_sum(val_t, key_t, par_t, ds_t, kn_t, pidx):
    mesh = plsc.VectorSubcoreMesh(core_axis_name="c", subcore_axis_name="s")
    cp = pltpu.CompilerParams()
    if "needs_layout_passes" in pltpu.CompilerParams.__dataclass_fields__:
        cp = dataclasses.replace(cp, needs_layout_passes=False)
    if "use_tc_tiling_on_sc" in pltpu.CompilerParams.__dataclass_fields__:
        cp = dataclasses.replace(cp, use_tc_tiling_on_sc=True)

    @functools.partial(
        pl.kernel,
        mesh=mesh,
        compiler_params=cp,
        out_type=jax.ShapeDtypeStruct((N2, 2 * D), jnp.float32),
        scratch_types=[
            pltpu.VMEM((NIDX,), jnp.int32),
            pltpu.VMEM((CHN, 2 * D), jnp.float32),
            pltpu.VMEM((CHN, 2 * D), jnp.float32),
            pltpu.VMEM((CHN, 2 * D), jnp.float32),
            pltpu.VMEM((CHN, 2 * D), jnp.float32),
            pltpu.VMEM((CHN, 2 * D), jnp.float32),
            pltpu.VMEM((CHN // 2, 2 * D), jnp.float32),
            pltpu.SemaphoreType.DMA,
        ],
    )
    def body(val_hbm, key_hbm, par_hbm, ds_hbm, kn_hbm, pidx_hbm, x_hbm,
             ibuf, vbuf, kbuf, pbuf, dbuf, nbuf, obuf, sem):
        wid = lax.axis_index("s") * 2 + lax.axis_index("c")

        @pl.loop(0, CPW)
        def _(c):
            cid = wid * CPW + c
            pltpu.sync_copy(pidx_hbm.at[cid], ibuf)
            pairs = [(val_hbm, vbuf), (key_hbm, kbuf), (par_hbm, pbuf),
                     (ds_hbm, dbuf), (kn_hbm, nbuf)]
            cps = []
            for g in range(CHN // NL):
                for j, (t, buf) in enumerate(pairs):
                    idxv = ibuf[pl.ds(j * CHN + g * NL, NL)]
                    cps.append(pltpu.async_copy(
                        t.at[idxv], buf.at[pl.ds(g * NL, NL)], sem))
            for cp_ in cps:
                cp_.wait()

            @pl.loop(0, CHN)
            def _(r):
                m = plsc.load_gather(ibuf, [jnp.full((NL,), 5 * CHN, jnp.int32) + r]) != 0
                pv = plsc.load_gather(ibuf, [jnp.full((NL,), 6 * CHN, jnp.int32) + r]) != 0
                pk = plsc.load_gather(ibuf, [jnp.full((NL,), 7 * CHN, jnp.int32) + r]) != 0
                pp = plsc.load_gather(ibuf, [jnp.full((NL,), 8 * CHN, jnp.int32) + r]) != 0
                r2 = r // 2
                half = (r % 2) * D
                for cc in range(D // NL):
                    lo = pl.ds(cc * NL, NL)
                    hi = pl.ds(D + cc * NL, NL)
                    v = jnp.where(pv, vbuf[r, hi], vbuf[r, lo])
                    k = jnp.where(pk, kbuf[r, hi], kbuf[r, lo])
                    p = jnp.where(pp, pbuf[r, hi], pbuf[r, lo])
                    t = jnp.where(m, k, v)
                    o = t + p + dbuf[r, lo] + nbuf[r, lo]
                    obuf[r2, pl.ds(half + cc * NL, NL)] = o

            pltpu.sync_copy(obuf, x_hbm.at[pl.ds(cid * (CHN // 2), CHN // 2)])

    return body(val_t, key_t, par_t, ds_t, kn_t, pidx)


def _ln_body(x_ref, g_ref, b_ref, o_ref):
    x = x_ref[...]
    g = g_ref[...]
    b = b_ref[...]
    outs = []
    for h in range(2):
        xh = x[:, h * D:(h + 1) * D]
        mu = jnp.mean(xh, axis=-1, keepdims=True)
        c = xh - mu
        var = jnp.mean(c * c, axis=-1, keepdims=True)
        outs.append(g * (c * lax.rsqrt(var + EPS)) + b)
    o_ref[...] = jnp.concatenate(outs, axis=-1)


_layernorm = pl.pallas_call(
    _ln_body,
    out_shape=jax.ShapeDtypeStruct((N2, 2 * D), jnp.float32),
    grid=(N2 // LNB,),
    in_specs=[
        pl.BlockSpec((LNB, 2 * D), lambda i: (i, 0)),
        pl.BlockSpec((1, D), lambda i: (0, 0)),
        pl.BlockSpec((1, D), lambda i: (0, 0)),
    ],
    out_specs=pl.BlockSpec((LNB, 2 * D), lambda i: (i, 0)),
)


def kernel(key_table, value_table, depth_table, sibling_table, node_type_table,
           parent_key_table, kind_table, ln_gamma, ln_beta,
           token_ids, node_types, depths, sibling_indices, parent_key_ids,
           kind_ids):
    tok = token_ids.reshape(-1)
    nt = node_types.reshape(-1)
    ival = jnp.clip(tok, 0, VAL_V - 1)
    ikey = jnp.clip(tok, 0, KEY_V - 1)
    ipar = jnp.clip(parent_key_ids.reshape(-1), 0, KEY_V - 1)
    ids = jnp.clip(depths.reshape(-1), 0, MAX_DEPTH - 1) * MAX_SIB + \
        jnp.clip(sibling_indices.reshape(-1), 0, MAX_SIB - 1)
    ikn = jnp.clip(kind_ids.reshape(-1), 0, KIND_V - 1) * NODE_TYPES + \
        jnp.clip(nt, 0, NODE_TYPES - 1)
    mask = ((nt == 0) | (nt == 2)).astype(jnp.int32)

    packed = jnp.stack([ival >> 1, ikey >> 1, ipar >> 1, ids, ikn,
                        mask, ival & 1, ikey & 1, ipar & 1], axis=0)
    packed = packed.reshape(9, NCHUNK, CHN).transpose(1, 0, 2).reshape(NCHUNK, NIDX)

    ds_small = (depth_table[:, None, :] + sibling_table[None, :, :]).reshape(-1, D)
    kn_small = (kind_table[:, None, :] + node_type_table[None, :, :]).reshape(-1, D)
    ds_t = jnp.concatenate([ds_small, ds_small], axis=1)
    kn_t = jnp.concatenate([kn_small, kn_small], axis=1)

    x = _sc_embed_sum(value_table.reshape(VAL_V // 2, 2 * D),
                      key_table.reshape(KEY_V // 2, 2 * D),
                      parent_key_table.reshape(KEY_V // 2, 2 * D),
                      ds_t, kn_t, packed)
    out = _layernorm(x, ln_gamma.reshape(1, D), ln_beta.reshape(1, D))
    return out.reshape(B, L, D)


# speedup vs baseline: 5.1507x; 5.1507x over previous
import dataclasses
import functools

import jax
import jax.numpy as jnp
from jax import lax
from jax.experimental import pallas as pl
from jax.experimental.pallas import tpu as pltpu
from jax.experimental.pallas import tpu_sc as plsc

D = 64
KEY_V = 100000
VAL_V = 1000000
MAX_DEPTH = 64
MAX_SIB = 256
NODE_TYPES = 4
KIND_V = 1000
B = 4096
L = 200
EPS = 1e-5

NL = 16
NW = 32
CHN = 128
N = B * L
N2 = N // 2
CPW = N // (NW * CHN)
NCHUNK = N // CHN
NIDX = 4 * CHN
NSMALL = MAX_DEPTH + MAX_SIB + NODE_TYPES + KIND_V
LNB = 4096
MASK16 = (1 << 16) - 1


def _sc_embed_sum(kv_t, par_t, smalls, pidx):
    mesh = plsc.VectorSubcoreMesh(core_axis_name="c", subcore_axis_name="s")
    cp = pltpu.CompilerParams()
    if "needs_layout_passes" in pltpu.CompilerParams.__dataclass_fields__:
        cp = dataclasses.replace(cp, needs_layout_passes=False)
    if "use_tc_tiling_on_sc" in pltpu.CompilerParams.__dataclass_fields__:
        cp = dataclasses.replace(cp, use_tc_tiling_on_sc=False)

    @functools.partial(
        pl.kernel,
        mesh=mesh,
        compiler_params=cp,
        out_type=jax.ShapeDtypeStruct((N2, 2 * D), jnp.float32),
        scratch_types=[
            pltpu.VMEM((NIDX,), jnp.int32),
            pltpu.VMEM((CHN, 32), jnp.int32),
            pltpu.VMEM((CHN, 32), jnp.int32),
            pltpu.VMEM((NSMALL, D), jnp.float32),
            pltpu.VMEM((CHN // 2, 2 * D), jnp.float32),
            pltpu.SemaphoreType.DMA,
        ],
    )
    def body(kv_hbm, par_hbm, smalls_hbm, pidx_hbm, x_hbm,
             ibuf, cbuf, pbuf, sbuf, obuf, sem):
        wid = lax.axis_index("s") * 2 + lax.axis_index("c")
        pltpu.sync_copy(smalls_hbm, sbuf)

        @pl.loop(0, CPW)
        def _(c):
            cid = wid * CPW + c
            pltpu.sync_copy(pidx_hbm.at[cid], ibuf)
            cps = []
            for g in range(CHN // NL):
                sl = pl.ds(g * NL, NL)
                cps.append(pltpu.async_copy(
                    kv_hbm.at[ibuf[pl.ds(g * NL, NL)]], cbuf.at[sl], sem))
                cps.append(pltpu.async_copy(
                    par_hbm.at[ibuf[pl.ds(CHN + g * NL, NL)]], pbuf.at[sl], sem))
            for cp_ in cps:
                cp_.wait()

            @pl.loop(0, CHN)
            def _(r):
                sp1 = plsc.load_gather(
                    ibuf, [jnp.full((NL,), 2 * CHN, jnp.int32) + r])
                sp2 = plsc.load_gather(
                    ibuf, [jnp.full((NL,), 3 * CHN, jnp.int32) + r])
                rd = sp1 & MASK16
                rs = lax.shift_right_logical(sp1, 16)
                rn = sp2 & MASK16
                rk = lax.shift_right_logical(sp2, 16)
                r2 = r // 2
                half = (r % 2) * D
                iota = lax.iota(jnp.int32, NL)
                w = [cbuf[r, pl.ds(0, NL)], cbuf[r, pl.ds(NL, NL)]]
                q = [pbuf[r, pl.ds(0, NL)], pbuf[r, pl.ds(NL, NL)]]
                for cc in range(D // NL):
                    wi, hi_half = cc // 2, cc % 2
                    if hi_half:
                        kv = plsc.bitcast(w[wi] & ~MASK16, jnp.float32)
                        pr = plsc.bitcast(q[wi] & ~MASK16, jnp.float32)
                    else:
                        kv = plsc.bitcast(lax.shift_left(w[wi], 16), jnp.float32)
                        pr = plsc.bitcast(lax.shift_left(q[wi], 16), jnp.float32)
                    col = cc * NL + iota
                    dv = plsc.load_gather(sbuf, [rd, col])
                    sv = plsc.load_gather(sbuf, [rs, col])
                    nv = plsc.load_gather(sbuf, [rn, col])
                    kv2 = plsc.load_gather(sbuf, [rk, col])
                    o = kv + pr + dv + sv + nv + kv2
                    obuf[r2, pl.ds(half + cc * NL, NL)] = o

            pltpu.sync_copy(obuf, x_hbm.at[pl.ds(cid * (CHN // 2), CHN // 2)])

    return body(kv_t, par_t, smalls, pidx)


def _ln_body(x_ref, g_ref, b_ref, o_ref):
    x = x_ref[...]
    g = g_ref[...]
    b = b_ref[...]
    outs = []
    for h in range(2):
        xh = x[:, h * D:(h + 1) * D]
        mu = jnp.mean(xh, axis=-1, keepdims=True)
        c = xh - mu
        var = jnp.mean(c * c, axis=-1, keepdims=True)
        outs.append(g * (c * lax.rsqrt(var + EPS)) + b)
    o_ref[...] = jnp.concatenate(outs, axis=-1)


_layernorm = pl.pallas_call(
    _ln_body,
    out_shape=jax.ShapeDtypeStruct((N2, 2 * D), jnp.float32),
    grid=(N2 // LNB,),
    in_specs=[
        pl.BlockSpec((LNB, 2 * D), lambda i: (i, 0)),
        pl.BlockSpec((1, D), lambda i: (0, 0)),
        pl.BlockSpec((1, D), lambda i: (0, 0)),
    ],
    out_specs=pl.BlockSpec((LNB, 2 * D), lambda i: (i, 0)),
)


def _swizzle_bf16(t):
    r = t.reshape(-1, 2, 2, NL).transpose(0, 1, 3, 2)
    b = r.astype(jnp.bfloat16).reshape(-1, 2)
    return jax.lax.bitcast_convert_type(b, jnp.int32).reshape(-1, 32)


def kernel(key_table, value_table, depth_table, sibling_table, node_type_table,
           parent_key_table, kind_table, ln_gamma, ln_beta,
           token_ids, node_types, depths, sibling_indices, parent_key_ids,
           kind_ids):
    tok = token_ids.reshape(-1)
    nt = node_types.reshape(-1)
    is_key = (nt == 0) | (nt == 2)
    ckv = jnp.where(is_key, jnp.clip(tok, 0, KEY_V - 1),
                    KEY_V + jnp.clip(tok, 0, VAL_V - 1))
    ipar = jnp.clip(parent_key_ids.reshape(-1), 0, KEY_V - 1)
    rd = jnp.clip(depths.reshape(-1), 0, MAX_DEPTH - 1)
    rs = MAX_DEPTH + jnp.clip(sibling_indices.reshape(-1), 0, MAX_SIB - 1)
    rn = MAX_DEPTH + MAX_SIB + jnp.clip(nt, 0, NODE_TYPES - 1)
    rk = MAX_DEPTH + MAX_SIB + NODE_TYPES + \
        jnp.clip(kind_ids.reshape(-1), 0, KIND_V - 1)
    p1 = rd | (rs << 16)
    p2 = rn | (rk << 16)

    packed = jnp.stack([ckv, ipar, p1, p2], axis=0)
    packed = packed.reshape(4, NCHUNK, CHN).transpose(1, 0, 2).reshape(NCHUNK, NIDX)

    kv_t = _swizzle_bf16(jnp.concatenate([key_table, value_table], axis=0))
    par_t = _swizzle_bf16(parent_key_table)
    smalls = jnp.concatenate(
        [depth_table, sibling_table, node_type_table, kind_table], axis=0)

    x = _sc_embed_sum(kv_t, par_t, smalls, packed)
    out = _layernorm(x, ln_gamma.reshape(1, D), ln_beta.reshape(1, D))
    return out.reshape(B, L, D)


# speedup vs baseline: 5.4319x; 1.0546x over previous
import dataclasses
import functools

import jax
import jax.numpy as jnp
from jax import lax
from jax.experimental import pallas as pl
from jax.experimental.pallas import tpu as pltpu
from jax.experimental.pallas import tpu_sc as plsc

D = 64
KEY_V = 100000
VAL_V = 1000000
MAX_DEPTH = 64
MAX_SIB = 256
NODE_TYPES = 4
KIND_V = 1000
B = 4096
L = 200
EPS = 1e-5

NL = 16
NW = 32
CHN = 128
N = B * L
N2 = N // 2
CPW = N // (NW * CHN)
NCHUNK = N // CHN
NIDX = 4 * CHN
NSMALL = MAX_DEPTH + MAX_SIB + NODE_TYPES + KIND_V
LNB = 4096
MASK16 = (1 << 16) - 1


def _sc_embed_sum(kv_t, par_t, smalls, pidx):
    mesh = plsc.VectorSubcoreMesh(core_axis_name="c", subcore_axis_name="s")
    cp = pltpu.CompilerParams()
    if "needs_layout_passes" in pltpu.CompilerParams.__dataclass_fields__:
        cp = dataclasses.replace(cp, needs_layout_passes=False)
    if "use_tc_tiling_on_sc" in pltpu.CompilerParams.__dataclass_fields__:
        cp = dataclasses.replace(cp, use_tc_tiling_on_sc=False)

    @functools.partial(
        pl.kernel,
        mesh=mesh,
        compiler_params=cp,
        out_type=jax.ShapeDtypeStruct((N2, D), jnp.int32),
        scratch_types=[
            pltpu.VMEM((NIDX,), jnp.int32),
            pltpu.VMEM((CHN, 32), jnp.int32),
            pltpu.VMEM((CHN, 32), jnp.int32),
            pltpu.VMEM((NSMALL, D), jnp.float32),
            pltpu.VMEM((CHN // 2, D), jnp.int32),
            pltpu.SemaphoreType.DMA,
        ],
    )
    def body(kv_hbm, par_hbm, smalls_hbm, pidx_hbm, x_hbm,
             ibuf, cbuf, pbuf, sbuf, obuf, sem):
        wid = lax.axis_index("s") * 2 + lax.axis_index("c")
        pltpu.sync_copy(smalls_hbm, sbuf)

        @pl.loop(0, CPW)
        def _(c):
            cid = wid * CPW + c
            pltpu.sync_copy(pidx_hbm.at[cid], ibuf)
            cps = []
            for g in range(CHN // NL):
                sl = pl.ds(g * NL, NL)
                cps.append(pltpu.async_copy(
                    kv_hbm.at[ibuf[pl.ds(g * NL, NL)]], cbuf.at[sl], sem))
                cps.append(pltpu.async_copy(
                    par_hbm.at[ibuf[pl.ds(CHN + g * NL, NL)]], pbuf.at[sl], sem))
            for cp_ in cps:
                cp_.wait()

            @pl.loop(0, CHN // 2)
            def _(r2):
                iota = lax.iota(jnp.int32, NL)
                toks = []
                for r in (2 * r2, 2 * r2 + 1):
                    sp1 = plsc.load_gather(
                        ibuf, [jnp.full((NL,), 2 * CHN, jnp.int32) + r])
                    sp2 = plsc.load_gather(
                        ibuf, [jnp.full((NL,), 3 * CHN, jnp.int32) + r])
                    rd = sp1 & MASK16
                    rs = lax.shift_right_logical(sp1, 16)
                    rn = sp2 & MASK16
                    rk = lax.shift_right_logical(sp2, 16)
                    w = [cbuf[r, pl.ds(0, NL)], cbuf[r, pl.ds(NL, NL)]]
                    q = [pbuf[r, pl.ds(0, NL)], pbuf[r, pl.ds(NL, NL)]]
                    vecs = []
                    for cc in range(D // NL):
                        wi, hi_half = cc // 2, cc % 2
                        if hi_half:
                            kv = plsc.bitcast(w[wi] & ~MASK16, jnp.float32)
                            pr = plsc.bitcast(q[wi] & ~MASK16, jnp.float32)
                        else:
                            kv = plsc.bitcast(lax.shift_left(w[wi], 16),
                                              jnp.float32)
                            pr = plsc.bitcast(lax.shift_left(q[wi], 16),
                                              jnp.float32)
                        col = cc * NL + iota
                        dv = plsc.load_gather(sbuf, [rd, col])
                        sv = plsc.load_gather(sbuf, [rs, col])
                        nv = plsc.load_gather(sbuf, [rn, col])
                        kv2 = plsc.load_gather(sbuf, [rk, col])
                        vecs.append(kv + pr + dv + sv + nv + kv2)
                    toks.append(vecs)
                for cc in range(D // NL):
                    be = plsc.bitcast(toks[0][cc], jnp.int32) + 0x8000
                    bo = plsc.bitcast(toks[1][cc], jnp.int32) + 0x8000
                    word = lax.shift_right_logical(be, 16) | (bo & ~MASK16)
                    obuf[r2, pl.ds(cc * NL, NL)] = word

            pltpu.sync_copy(obuf, x_hbm.at[pl.ds(cid * (CHN // 2), CHN // 2)])

    return body(kv_t, par_t, smalls, pidx)


def _ln1(xh, g, b):
    mu = jnp.mean(xh, axis=-1, keepdims=True)
    c = xh - mu
    var = jnp.mean(c * c, axis=-1, keepdims=True)
    return g * (c * lax.rsqrt(var + EPS)) + b


def _ln_body(x_ref, g_ref, b_ref, o_ref):
    xi = x_ref[...]
    g = g_ref[...]
    b = b_ref[...]
    lo = lax.bitcast_convert_type(lax.shift_left(xi, 16), jnp.float32)
    hi = lax.bitcast_convert_type(xi & ~MASK16, jnp.float32)
    ys = []
    for h in range(2):
        sl = slice(h * D, (h + 1) * D)
        ys.append(_ln1(lo[:, sl], g, b))
        ys.append(_ln1(hi[:, sl], g, b))
    y = jnp.stack([ys[0], ys[1], ys[2], ys[3]], axis=1)
    o_ref[...] = y.reshape(4 * LNB, D)


N4 = N2 // 2
_layernorm = pl.pallas_call(
    _ln_body,
    out_shape=jax.ShapeDtypeStruct((N, D), jnp.float32),
    grid=(N4 // LNB,),
    in_specs=[
        pl.BlockSpec((LNB, 2 * D), lambda i: (i, 0)),
        pl.BlockSpec((1, D), lambda i: (0, 0)),
        pl.BlockSpec((1, D), lambda i: (0, 0)),
    ],
    out_specs=pl.BlockSpec((4 * LNB, D), lambda i: (i, 0)),
)


def _swizzle_bf16(t):
    r = t.reshape(-1, 2, 2, NL).transpose(0, 1, 3, 2)
    b = r.astype(jnp.bfloat16).reshape(-1, 2)
    return jax.lax.bitcast_convert_type(b, jnp.int32).reshape(-1, 32)


def kernel(key_table, value_table, depth_table, sibling_table, node_type_table,
           parent_key_table, kind_table, ln_gamma, ln_beta,
           token_ids, node_types, depths, sibling_indices, parent_key_ids,
           kind_ids):
    tok = token_ids.reshape(-1)
    nt = node_types.reshape(-1)
    is_key = (nt == 0) | (nt == 2)
    ckv = jnp.where(is_key, jnp.clip(tok, 0, KEY_V - 1),
                    KEY_V + jnp.clip(tok, 0, VAL_V - 1))
    ipar = jnp.clip(parent_key_ids.reshape(-1), 0, KEY_V - 1)
    rd = jnp.clip(depths.reshape(-1), 0, MAX_DEPTH - 1)
    rs = MAX_DEPTH + jnp.clip(sibling_indices.reshape(-1), 0, MAX_SIB - 1)
    rn = MAX_DEPTH + MAX_SIB + jnp.clip(nt, 0, NODE_TYPES - 1)
    rk = MAX_DEPTH + MAX_SIB + NODE_TYPES + \
        jnp.clip(kind_ids.reshape(-1), 0, KIND_V - 1)
    p1 = rd | (rs << 16)
    p2 = rn | (rk << 16)

    packed = jnp.stack([ckv, ipar, p1, p2], axis=0)
    packed = packed.reshape(4, NCHUNK, CHN).transpose(1, 0, 2).reshape(NCHUNK, NIDX)

    kv_t = _swizzle_bf16(jnp.concatenate([key_table, value_table], axis=0))
    par_t = _swizzle_bf16(parent_key_table)
    smalls = jnp.concatenate(
        [depth_table, sibling_table, node_type_table, kind_table], axis=0)

    x = _sc_embed_sum(kv_t, par_t, smalls, packed)
    out = _layernorm(x.reshape(N4, 2 * D),
                     ln_gamma.reshape(1, D), ln_beta.reshape(1, D))
    return out.reshape(B, L, D)
